# Initial kernel scaffold; baseline (speedup 1.0000x reference)
import jax
import jax.numpy as jnp
from jax.experimental import pallas as pl


def kernel(dense, onehot, multihot_0, multihot_1, fm_w, fm_emb, nn_w0, nn_b0, nn_w1, nn_b1, concat_w, concat_b):
    raise NotImplementedError("write your pallas kernel here")


# speedup vs baseline: 2.2214x; 2.2214x over previous
import functools

import jax
import jax.numpy as jnp
from jax import lax
from jax.experimental import pallas as pl
from jax.experimental.pallas import tpu as pltpu
from jax.experimental.pallas import tpu_sc as plsc

_B = 16384
_V = 1000000
_E = 16
_OH = 26
_MH = 20

_NC = 2
_NS = 16
_NW = _NC * _NS
_BPW = _B // _NW
_C = 64
_NCHUNK = _BPW // _C


def _sc_body(oh_hbm, mh0_hbm, mh1_hbm, emb_hbm, fmw_hbm,
             out_oh, out_mh, out_fmw,
             oh_idx, mh0_idx, mh1_idx, oh_rows, mh0_rows, mh1_rows,
             fmw_vals, means, sem0, sem1, sem2, sem3):
    wid = lax.axis_index("s") * _NC + lax.axis_index("c")

    def chunk_body(c, carry):
        base = wid * _BPW + c * _C
        pltpu.sync_copy(oh_hbm.at[pl.ds(base * _OH, _C * _OH)], oh_idx)
        pltpu.sync_copy(mh0_hbm.at[pl.ds(base * _MH, _C * _MH)], mh0_idx)
        pltpu.sync_copy(mh1_hbm.at[pl.ds(base * _MH, _C * _MH)], mh1_idx)
        cp_oh = pltpu.async_copy(emb_hbm.at[oh_idx], oh_rows, sem0)
        cp_m0 = pltpu.async_copy(emb_hbm.at[mh0_idx], mh0_rows, sem1)
        cp_m1 = pltpu.async_copy(emb_hbm.at[mh1_idx], mh1_rows, sem2)
        cp_fw = pltpu.async_copy(fmw_hbm.at[oh_idx], fmw_vals, sem3)
        cp_m0.wait()
        cp_m1.wait()

        def mean_body(i, carry2):
            off = i * _MH
            acc0 = mh0_rows[off]
            acc1 = mh1_rows[off]
            for j in range(1, _MH):
                acc0 = acc0 + mh0_rows[off + j]
                acc1 = acc1 + mh1_rows[off + j]
            means[pl.ds(i * 2 * _E, _E)] = acc0 * (1.0 / _MH)
            means[pl.ds(i * 2 * _E + _E, _E)] = acc1 * (1.0 / _MH)
            return carry2

        lax.fori_loop(0, _C, mean_body, 0, unroll=2)
        pltpu.sync_copy(means, out_mh.at[pl.ds(base * 2 * _E, _C * 2 * _E)])
        cp_oh.wait()
        pltpu.sync_copy(oh_rows, out_oh.at[pl.ds(base * _OH, _C * _OH)])
        cp_fw.wait()
        pltpu.sync_copy(fmw_vals, out_fmw.at[pl.ds(base * _OH, _C * _OH)])
        return carry

    lax.fori_loop(0, _NCHUNK, chunk_body, 0)


def _sc_gather(onehot_flat, mh0_flat, mh1_flat, fm_emb, fm_w_flat):
    kern = pl.kernel(
        _sc_body,
        out_type=[
            jax.ShapeDtypeStruct((_B * _OH, _E), jnp.float32),
            jax.ShapeDtypeStruct((_B * 2 * _E,), jnp.float32),
            jax.ShapeDtypeStruct((_B * _OH,), jnp.float32),
        ],
        mesh=plsc.VectorSubcoreMesh(core_axis_name="c", subcore_axis_name="s"),
        compiler_params=pltpu.CompilerParams(use_tc_tiling_on_sc=False),
        scratch_types=[
            pltpu.VMEM((_C * _OH,), jnp.int32),
            pltpu.VMEM((_C * _MH,), jnp.int32),
            pltpu.VMEM((_C * _MH,), jnp.int32),
            pltpu.VMEM((_C * _OH, _E), jnp.float32),
            pltpu.VMEM((_C * _MH, _E), jnp.float32),
            pltpu.VMEM((_C * _MH, _E), jnp.float32),
            pltpu.VMEM((_C * _OH,), jnp.float32),
            pltpu.VMEM((_C * 2 * _E,), jnp.float32),
            pltpu.SemaphoreType.DMA,
            pltpu.SemaphoreType.DMA,
            pltpu.SemaphoreType.DMA,
            pltpu.SemaphoreType.DMA,
        ],
    )
    return kern(onehot_flat, mh0_flat, mh1_flat, fm_emb, fm_w_flat)


def _tc_body(oh_ref, mh_ref, fw_ref, dn_ref, s416_ref, s32_ref,
             w0o_ref, w0m_ref, w0d_ref, b0_ref, w1_ref, b1_ref,
             fwh_ref, cw2_ref, cwn_ref, cb_ref, out_ref):
    f32 = jnp.float32

    def dot(a, b):
        return lax.dot_general(a, b, (((1,), (0,)), ((), ())),
                               preferred_element_type=f32)

    x = oh_ref[...]
    m = mh_ref[...]
    fw = fw_ref[...]
    dn = dn_ref[...]
    s = dot(x, s416_ref[...]) + dot(m, s32_ref[...])
    sq = dot(x * x, s416_ref[...]) + dot(m * m, s32_ref[...])
    fm2 = 0.5 * (s * s - sq)
    h0 = jnp.maximum(
        dot(x, w0o_ref[...]) + dot(m, w0m_ref[...]) + dot(dn, w0d_ref[...])
        + b0_ref[...], 0.0)
    h1 = jnp.maximum(dot(h0, w1_ref[...]) + b1_ref[...], 0.0)
    z = dot(fw, fwh_ref[...]) + dot(fm2, cw2_ref[...]) + dot(h1, cwn_ref[...])
    z = z + cb_ref[...]
    out_ref[...] = 1.0 / (1.0 + jnp.exp(-z))


def _tc_head(oh_flat, mh_flat, fw_vals, dense, s416, s32, w0o, w0m, w0d,
             b0, w1, b1, fwh, cw2, cwn, cb):
    bb = 2048
    grid = (_B // bb,)

    def row_block(minor):
        return pl.BlockSpec((bb, minor), lambda i: (i, 0))

    def full_block(shape):
        return pl.BlockSpec(shape, lambda i: (0, 0))

    return pl.pallas_call(
        _tc_body,
        grid=grid,
        in_specs=[
            row_block(_OH * _E),
            row_block(2 * _E),
            row_block(_OH),
            row_block(13),
            full_block((_OH * _E, _E)),
            full_block((2 * _E, _E)),
            full_block((_OH * _E, 64)),
            full_block((2 * _E, 64)),
            full_block((13, 64)),
            full_block((1, 64)),
            full_block((64, 12)),
            full_block((1, 12)),
            full_block((_OH, 1)),
            full_block((_E, 1)),
            full_block((12, 1)),
            full_block((1, 1)),
        ],
        out_specs=pl.BlockSpec((bb, 1), lambda i: (i, 0)),
        out_shape=jax.ShapeDtypeStruct((_B, 1), jnp.float32),
    )(oh_flat, mh_flat, fw_vals, dense, s416, s32, w0o, w0m, w0d, b0, w1,
      b1, fwh, cw2, cwn, cb)


def kernel(dense, onehot, multihot_0, multihot_1, fm_w, fm_emb,
           nn_w0, nn_b0, nn_w1, nn_b1, concat_w, concat_b):
    oh_rows, mh_mean, fw_vals = _sc_gather(
        onehot.reshape(-1).astype(jnp.int32),
        multihot_0.reshape(-1).astype(jnp.int32),
        multihot_1.reshape(-1).astype(jnp.int32),
        fm_emb,
        fm_w.reshape(-1),
    )
    oh_flat = oh_rows.reshape(_B, _OH * _E)
    mh_flat = mh_mean.reshape(_B, 2 * _E)
    fw_flat = fw_vals.reshape(_B, _OH)

    eye = jnp.eye(_E, dtype=jnp.float32)
    s416 = jnp.tile(eye, (_OH, 1))
    s32 = jnp.tile(eye, (2, 1))
    w0o = nn_w0[: _OH * _E]
    w0m = nn_w0[_OH * _E: (_OH + 2) * _E]
    w0d = nn_w0[(_OH + 2) * _E:]
    fwh = jnp.full((_OH, 1), concat_w[0, 0], dtype=jnp.float32)
    cw2 = concat_w[1: 1 + _E]
    cwn = concat_w[1 + _E:]
    return _tc_head(oh_flat, mh_flat, fw_flat, dense, s416, s32, w0o, w0m,
                    w0d, nn_b0.reshape(1, 64), nn_w1, nn_b1.reshape(1, 12),
                    fwh, cw2, cwn, concat_b.reshape(1, 1))


# speedup vs baseline: 4.5839x; 2.0635x over previous
import functools

import jax
import jax.numpy as jnp
from jax import lax
from jax.experimental import pallas as pl
from jax.experimental.pallas import tpu as pltpu
from jax.experimental.pallas import tpu_sc as plsc

_B = 16384
_V = 1000000
_E = 16
_OH = 26
_MH = 20

_NC = 2
_NS = 16
_NW = _NC * _NS
_BPW = _B // _NW
_C = 64
_NCHUNK = _BPW // _C


_TBLK = 512
_TNB = (_V + _TBLK - 1) // _TBLK
_TFULL = _V // _TBLK
_TTAIL = _V - _TFULL * _TBLK


def _tr_body(tabT, tailT, out1d, in0, in1, tin, ob0, ob1, si0, si1, so0, so1):
    wid = lax.axis_index("s") * _NC + lax.axis_index("c")
    iota = lax.iota(jnp.int32, 16)
    f_idx = [iota * 16 + f for f in range(16)]
    inbufs = (in0, in1)
    obufs = (ob0, ob1)
    isems = (si0, si1)
    osems = (so0, so1)

    def issue_in(blk, buf, sem):
        @pl.when(blk < _TFULL)
        def _():
            pltpu.async_copy(tabT.at[:, pl.ds(blk * _TBLK, _TBLK)], buf, sem)

    def scatter_block(inbuf, ob):
        def g_body(g, carry):
            for f in range(16):
                v = inbuf[f, pl.ds(g * 16, 16)]
                plsc.store_scatter(ob, [f_idx[f] + g * 256], v)
            return carry
        lax.fori_loop(0, _TBLK // 16, g_body, 0)

    issue_in(wid, in0, si0)
    issue_in(wid + _NW, in1, si1)

    def outer(i2, carry):
        for b in range(2):
            i = 2 * i2 + b
            blk = wid + _NW * i

            @pl.when(blk < _TFULL)
            def _():
                pltpu.make_async_copy(
                    tabT.at[:, pl.ds(0, _TBLK)], inbufs[b], isems[b]).wait()

                @pl.when(i2 >= 1)
                def _():
                    pltpu.make_async_copy(
                        obufs[b], out1d.at[pl.ds(0, _TBLK * 16)],
                        osems[b]).wait()

                scatter_block(inbufs[b], obufs[b])
                pltpu.async_copy(
                    obufs[b], out1d.at[pl.ds(blk * _TBLK * 16, _TBLK * 16)],
                    osems[b])
                issue_in(blk + 2 * _NW, inbufs[b], isems[b])
        return carry

    lax.fori_loop(0, 31, outer, 0)
    for b in range(2):
        pltpu.make_async_copy(
            obufs[b], out1d.at[pl.ds(0, _TBLK * 16)], osems[b]).wait()

    @pl.when(wid == 0)
    def _():
        pltpu.sync_copy(tailT, tin)

        def tg_body(g, carry):
            for f in range(16):
                v = tin[f, pl.ds(g * 16, 16)]
                plsc.store_scatter(ob0, [f_idx[f] + g * 256], v)
            return carry
        lax.fori_loop(0, 128 // 16, tg_body, 0)
        pltpu.sync_copy(
            ob0.at[pl.ds(0, 128 * 16)],
            out1d.at[pl.ds((_V - 128) * 16, 128 * 16)])


def _linearize_table(tabT):
    kern = pl.kernel(
        _tr_body,
        out_type=[jax.ShapeDtypeStruct((_V * _E,), jnp.float32)],
        mesh=plsc.VectorSubcoreMesh(core_axis_name="c", subcore_axis_name="s"),
        compiler_params=pltpu.CompilerParams(
            use_tc_tiling_on_sc=True, needs_layout_passes=False),
        scratch_types=[
            pltpu.VMEM((16, _TBLK), jnp.float32),
            pltpu.VMEM((16, _TBLK), jnp.float32),
            pltpu.VMEM((16, 128), jnp.float32),
            pltpu.VMEM((_TBLK * 16,), jnp.float32),
            pltpu.VMEM((_TBLK * 16,), jnp.float32),
            pltpu.SemaphoreType.DMA,
            pltpu.SemaphoreType.DMA,
            pltpu.SemaphoreType.DMA,
            pltpu.SemaphoreType.DMA,
        ],
    )
    return kern(tabT, tabT[:, _V - 128:])[0]


def _sc_body(oh_hbm, mh0_hbm, mh1_hbm, emb_hbm, fmw_hbm,
             out_oh, out_mh, out_fmw,
             oh_idx, mh0_idx, mh1_idx, oh_rows, mh0_rows, mh1_rows,
             fmw_vals, means, sem0, sem1, sem2, sem3):
    wid = lax.axis_index("s") * _NC + lax.axis_index("c")

    def chunk_body(c, carry):
        base = wid * _BPW + c * _C
        pltpu.sync_copy(oh_hbm.at[pl.ds(base * _OH, _C * _OH)], oh_idx)
        pltpu.sync_copy(mh0_hbm.at[pl.ds(base * _MH, _C * _MH)], mh0_idx)
        pltpu.sync_copy(mh1_hbm.at[pl.ds(base * _MH, _C * _MH)], mh1_idx)
        cp_oh = pltpu.async_copy(emb_hbm.at[oh_idx], oh_rows, sem0)
        cp_m0 = pltpu.async_copy(emb_hbm.at[mh0_idx], mh0_rows, sem1)
        cp_m1 = pltpu.async_copy(emb_hbm.at[mh1_idx], mh1_rows, sem2)
        cp_fw = pltpu.async_copy(fmw_hbm.at[oh_idx], fmw_vals, sem3)
        cp_m0.wait()
        cp_m1.wait()

        def mean_body(i, carry2):
            off = i * _MH
            acc0 = mh0_rows[off]
            acc1 = mh1_rows[off]
            for j in range(1, _MH):
                acc0 = acc0 + mh0_rows[off + j]
                acc1 = acc1 + mh1_rows[off + j]
            means[pl.ds(i * 2 * _E, _E)] = acc0 * (1.0 / _MH)
            means[pl.ds(i * 2 * _E + _E, _E)] = acc1 * (1.0 / _MH)
            return carry2

        lax.fori_loop(0, _C, mean_body, 0, unroll=2)
        pltpu.sync_copy(means, out_mh.at[pl.ds(base * 2 * _E, _C * 2 * _E)])
        cp_oh.wait()
        pltpu.sync_copy(oh_rows, out_oh.at[pl.ds(base * _OH, _C * _OH)])
        cp_fw.wait()
        pltpu.sync_copy(fmw_vals, out_fmw.at[pl.ds(base * _OH, _C * _OH)])
        return carry

    lax.fori_loop(0, _NCHUNK, chunk_body, 0)


def _sc_gather(onehot_flat, mh0_flat, mh1_flat, fm_emb, fm_w_flat):
    kern = pl.kernel(
        _sc_body,
        out_type=[
            jax.ShapeDtypeStruct((_B * _OH, _E), jnp.float32),
            jax.ShapeDtypeStruct((_B * 2 * _E,), jnp.float32),
            jax.ShapeDtypeStruct((_B * _OH,), jnp.float32),
        ],
        mesh=plsc.VectorSubcoreMesh(core_axis_name="c", subcore_axis_name="s"),
        compiler_params=pltpu.CompilerParams(use_tc_tiling_on_sc=False),
        scratch_types=[
            pltpu.VMEM((_C * _OH,), jnp.int32),
            pltpu.VMEM((_C * _MH,), jnp.int32),
            pltpu.VMEM((_C * _MH,), jnp.int32),
            pltpu.VMEM((_C * _OH, _E), jnp.float32),
            pltpu.VMEM((_C * _MH, _E), jnp.float32),
            pltpu.VMEM((_C * _MH, _E), jnp.float32),
            pltpu.VMEM((_C * _OH,), jnp.float32),
            pltpu.VMEM((_C * 2 * _E,), jnp.float32),
            pltpu.SemaphoreType.DMA,
            pltpu.SemaphoreType.DMA,
            pltpu.SemaphoreType.DMA,
            pltpu.SemaphoreType.DMA,
        ],
    )
    return kern(onehot_flat, mh0_flat, mh1_flat, fm_emb, fm_w_flat)


def _tc_body(oh_ref, mh_ref, fw_ref, dn_ref, s416_ref, s32_ref,
             w0o_ref, w0m_ref, w0d_ref, b0_ref, w1_ref, b1_ref,
             fwh_ref, cw2_ref, cwn_ref, cb_ref, out_ref):
    f32 = jnp.float32

    def dot(a, b):
        return lax.dot_general(a, b, (((1,), (0,)), ((), ())),
                               preferred_element_type=f32)

    x = oh_ref[...]
    m = mh_ref[...]
    fw = fw_ref[...]
    dn = dn_ref[...]
    s = dot(x, s416_ref[...]) + dot(m, s32_ref[...])
    sq = dot(x * x, s416_ref[...]) + dot(m * m, s32_ref[...])
    fm2 = 0.5 * (s * s - sq)
    h0 = jnp.maximum(
        dot(x, w0o_ref[...]) + dot(m, w0m_ref[...]) + dot(dn, w0d_ref[...])
        + b0_ref[...], 0.0)
    h1 = jnp.maximum(dot(h0, w1_ref[...]) + b1_ref[...], 0.0)
    z = dot(fw, fwh_ref[...]) + dot(fm2, cw2_ref[...]) + dot(h1, cwn_ref[...])
    z = z + cb_ref[...]
    out_ref[...] = 1.0 / (1.0 + jnp.exp(-z))


def _tc_head(oh_flat, mh_flat, fw_vals, dense, s416, s32, w0o, w0m, w0d,
             b0, w1, b1, fwh, cw2, cwn, cb):
    bb = 2048
    grid = (_B // bb,)

    def row_block(minor):
        return pl.BlockSpec((bb, minor), lambda i: (i, 0))

    def full_block(shape):
        return pl.BlockSpec(shape, lambda i: (0, 0))

    return pl.pallas_call(
        _tc_body,
        grid=grid,
        in_specs=[
            row_block(_OH * _E),
            row_block(2 * _E),
            row_block(_OH),
            row_block(13),
            full_block((_OH * _E, _E)),
            full_block((2 * _E, _E)),
            full_block((_OH * _E, 64)),
            full_block((2 * _E, 64)),
            full_block((13, 64)),
            full_block((1, 64)),
            full_block((64, 12)),
            full_block((1, 12)),
            full_block((_OH, 1)),
            full_block((_E, 1)),
            full_block((12, 1)),
            full_block((1, 1)),
        ],
        out_specs=pl.BlockSpec((bb, 1), lambda i: (i, 0)),
        out_shape=jax.ShapeDtypeStruct((_B, 1), jnp.float32),
    )(oh_flat, mh_flat, fw_vals, dense, s416, s32, w0o, w0m, w0d, b0, w1,
      b1, fwh, cw2, cwn, cb)


def kernel(dense, onehot, multihot_0, multihot_1, fm_w, fm_emb,
           nn_w0, nn_b0, nn_w1, nn_b1, concat_w, concat_b):
    tab_lin = _linearize_table(fm_emb.T)
    oh_rows, mh_mean, fw_vals = _sc_gather(
        onehot.reshape(-1).astype(jnp.int32),
        multihot_0.reshape(-1).astype(jnp.int32),
        multihot_1.reshape(-1).astype(jnp.int32),
        tab_lin.reshape(_V, _E),
        fm_w.reshape(-1),
    )
    oh_flat = oh_rows.reshape(_B, _OH * _E)
    mh_flat = mh_mean.reshape(_B, 2 * _E)
    fw_flat = fw_vals.reshape(_B, _OH)

    eye = jnp.eye(_E, dtype=jnp.float32)
    s416 = jnp.tile(eye, (_OH, 1))
    s32 = jnp.tile(eye, (2, 1))
    w0o = nn_w0[: _OH * _E]
    w0m = nn_w0[_OH * _E: (_OH + 2) * _E]
    w0d = nn_w0[(_OH + 2) * _E:]
    fwh = jnp.full((_OH, 1), concat_w[0, 0], dtype=jnp.float32)
    cw2 = concat_w[1: 1 + _E]
    cwn = concat_w[1 + _E:]
    return _tc_head(oh_flat, mh_flat, fw_flat, dense, s416, s32, w0o, w0m,
                    w0d, nn_b0.reshape(1, 64), nn_w1, nn_b1.reshape(1, 12),
                    fwh, cw2, cwn, concat_b.reshape(1, 1))


# speedup vs baseline: 4.6422x; 1.0127x over previous
import functools

import jax
import jax.numpy as jnp
from jax import lax
from jax.experimental import pallas as pl
from jax.experimental.pallas import tpu as pltpu
from jax.experimental.pallas import tpu_sc as plsc

_B = 16384
_V = 1000000
_E = 16
_OH = 26
_MH = 20

_NC = 2
_NS = 16
_NW = _NC * _NS
_BPW = _B // _NW
_C = 32
_NCHUNK = _BPW // _C


_TBLK = 1024
_TFULL = _V // _TBLK
_TTAIL = 640


def _tr_body(tabT, tailT, out1d, in0, in1, tin, ob0, ob1, si0, si1, so0, so1):
    wid = lax.axis_index("s") * _NC + lax.axis_index("c")
    iota = lax.iota(jnp.int32, 16)
    f_idx = [iota * 16 + f for f in range(16)]
    inbufs = (in0, in1)
    obufs = (ob0, ob1)
    isems = (si0, si1)
    osems = (so0, so1)

    def issue_in(blk, buf, sem):
        @pl.when(blk < _TFULL)
        def _():
            pltpu.async_copy(tabT.at[:, pl.ds(blk * _TBLK, _TBLK)], buf, sem)

    def scatter_block(inbuf, ob):
        def g_body(g, carry):
            for f in range(16):
                v = inbuf[f, pl.ds(g * 16, 16)]
                plsc.store_scatter(ob, [f_idx[f] + g * 256], v)
            return carry
        lax.fori_loop(0, _TBLK // 16, g_body, 0)

    issue_in(wid, in0, si0)
    issue_in(wid + _NW, in1, si1)

    def outer(i2, carry):
        for b in range(2):
            i = 2 * i2 + b
            blk = wid + _NW * i

            @pl.when(blk < _TFULL)
            def _():
                pltpu.make_async_copy(
                    tabT.at[:, pl.ds(0, _TBLK)], inbufs[b], isems[b]).wait()

                @pl.when(i2 >= 1)
                def _():
                    pltpu.make_async_copy(
                        obufs[b], out1d.at[pl.ds(0, _TBLK * 16)],
                        osems[b]).wait()

                scatter_block(inbufs[b], obufs[b])
                pltpu.async_copy(
                    obufs[b], out1d.at[pl.ds(blk * _TBLK * 16, _TBLK * 16)],
                    osems[b])
                issue_in(blk + 2 * _NW, inbufs[b], isems[b])
        return carry

    lax.fori_loop(0, 16, outer, 0)
    for b in range(2):
        pltpu.make_async_copy(
            obufs[b], out1d.at[pl.ds(0, _TBLK * 16)], osems[b]).wait()

    @pl.when(wid == 0)
    def _():
        pltpu.sync_copy(tailT, tin)

        def tg_body(g, carry):
            for f in range(16):
                v = tin[f, pl.ds(g * 16, 16)]
                plsc.store_scatter(ob0, [f_idx[f] + g * 256], v)
            return carry
        lax.fori_loop(0, _TTAIL // 16, tg_body, 0)
        pltpu.sync_copy(
            ob0.at[pl.ds(0, _TTAIL * 16)],
            out1d.at[pl.ds((_V - _TTAIL) * 16, _TTAIL * 16)])


def _linearize_table(tabT):
    kern = pl.kernel(
        _tr_body,
        out_type=[jax.ShapeDtypeStruct((_V * _E,), jnp.float32)],
        mesh=plsc.VectorSubcoreMesh(core_axis_name="c", subcore_axis_name="s"),
        compiler_params=pltpu.CompilerParams(
            use_tc_tiling_on_sc=True, needs_layout_passes=False),
        scratch_types=[
            pltpu.VMEM((16, _TBLK), jnp.float32),
            pltpu.VMEM((16, _TBLK), jnp.float32),
            pltpu.VMEM((16, _TTAIL), jnp.float32),
            pltpu.VMEM((_TBLK * 16,), jnp.float32),
            pltpu.VMEM((_TBLK * 16,), jnp.float32),
            pltpu.SemaphoreType.DMA,
            pltpu.SemaphoreType.DMA,
            pltpu.SemaphoreType.DMA,
            pltpu.SemaphoreType.DMA,
        ],
    )
    return kern(tabT, tabT[:, _V - _TTAIL:])[0]


def _sc_body(oh_hbm, mh0_hbm, mh1_hbm, emb_hbm, fmw_hbm,
             out_oh, out_mh, out_fmw,
             oh_idx0, oh_idx1, mh0_idx0, mh0_idx1, mh1_idx0, mh1_idx1,
             oh_rows0, oh_rows1, mh0_rows0, mh0_rows1, mh1_rows0, mh1_rows1,
             fmw0, fmw1, means0, means1, sg0, sg1, so0, so1):
    wid = lax.axis_index("s") * _NC + lax.axis_index("c")
    oh_idx = (oh_idx0, oh_idx1)
    mh0_idx = (mh0_idx0, mh0_idx1)
    mh1_idx = (mh1_idx0, mh1_idx1)
    oh_rows = (oh_rows0, oh_rows1)
    mh0_rows = (mh0_rows0, mh0_rows1)
    mh1_rows = (mh1_rows0, mh1_rows1)
    fmw = (fmw0, fmw1)
    means = (means0, means1)
    sg = (sg0, sg1)
    so = (so0, so1)

    def stage_in(c, b):
        base = wid * _BPW + c * _C
        pltpu.sync_copy(oh_hbm.at[pl.ds(base * _OH, _C * _OH)], oh_idx[b])
        pltpu.sync_copy(mh0_hbm.at[pl.ds(base * _MH, _C * _MH)], mh0_idx[b])
        pltpu.sync_copy(mh1_hbm.at[pl.ds(base * _MH, _C * _MH)], mh1_idx[b])
        pltpu.async_copy(emb_hbm.at[oh_idx[b]], oh_rows[b], sg[b])
        pltpu.async_copy(emb_hbm.at[mh0_idx[b]], mh0_rows[b], sg[b])
        pltpu.async_copy(emb_hbm.at[mh1_idx[b]], mh1_rows[b], sg[b])
        pltpu.async_copy(fmw_hbm.at[oh_idx[b]], fmw[b], sg[b])

    def wait_gathers(b):
        pltpu.make_async_copy(emb_hbm.at[oh_idx[b]], oh_rows[b], sg[b]).wait()
        pltpu.make_async_copy(emb_hbm.at[mh0_idx[b]], mh0_rows[b],
                              sg[b]).wait()
        pltpu.make_async_copy(emb_hbm.at[mh1_idx[b]], mh1_rows[b],
                              sg[b]).wait()
        pltpu.make_async_copy(fmw_hbm.at[oh_idx[b]], fmw[b], sg[b]).wait()

    def wait_outs(b):
        pltpu.make_async_copy(oh_rows[b], out_oh.at[pl.ds(0, _C * _OH)],
                              so[b]).wait()
        pltpu.make_async_copy(means[b], out_mh.at[pl.ds(0, _C * 2 * _E)],
                              so[b]).wait()
        pltpu.make_async_copy(fmw[b], out_fmw.at[pl.ds(0, _C * _OH)],
                              so[b]).wait()

    stage_in(0, 0)
    stage_in(1, 1)

    def outer(c2, carry):
        for b in range(2):
            c = 2 * c2 + b
            base = wid * _BPW + c * _C
            wait_gathers(b)

            def mean_body(i, carry2):
                off = i * _MH
                acc0 = mh0_rows[b][off]
                acc1 = mh1_rows[b][off]
                for j in range(1, _MH):
                    acc0 = acc0 + mh0_rows[b][off + j]
                    acc1 = acc1 + mh1_rows[b][off + j]
                means[b][pl.ds(i * 2 * _E, _E)] = acc0 * (1.0 / _MH)
                means[b][pl.ds(i * 2 * _E + _E, _E)] = acc1 * (1.0 / _MH)
                return carry2

            lax.fori_loop(0, _C, mean_body, 0, unroll=2)

            pltpu.async_copy(oh_rows[b],
                             out_oh.at[pl.ds(base * _OH, _C * _OH)], so[b])
            pltpu.async_copy(means[b],
                             out_mh.at[pl.ds(base * 2 * _E, _C * 2 * _E)],
                             so[b])
            pltpu.async_copy(fmw[b],
                             out_fmw.at[pl.ds(base * _OH, _C * _OH)], so[b])
            wait_outs(b)

            @pl.when(c + 2 < _NCHUNK)
            def _():
                stage_in(c + 2, b)
        return carry

    lax.fori_loop(0, _NCHUNK // 2, outer, 0)


def _sc_gather(onehot_flat, mh0_flat, mh1_flat, fm_emb, fm_w_flat):
    kern = pl.kernel(
        _sc_body,
        out_type=[
            jax.ShapeDtypeStruct((_B * _OH, _E), jnp.float32),
            jax.ShapeDtypeStruct((_B * 2 * _E,), jnp.float32),
            jax.ShapeDtypeStruct((_B * _OH,), jnp.float32),
        ],
        mesh=plsc.VectorSubcoreMesh(core_axis_name="c", subcore_axis_name="s"),
        compiler_params=pltpu.CompilerParams(use_tc_tiling_on_sc=False),
        scratch_types=(
            [pltpu.VMEM((_C * _OH,), jnp.int32)] * 2
            + [pltpu.VMEM((_C * _MH,), jnp.int32)] * 4
            + [pltpu.VMEM((_C * _OH, _E), jnp.float32)] * 2
            + [pltpu.VMEM((_C * _MH, _E), jnp.float32)] * 4
            + [pltpu.VMEM((_C * _OH,), jnp.float32)] * 2
            + [pltpu.VMEM((_C * 2 * _E,), jnp.float32)] * 2
            + [pltpu.SemaphoreType.DMA] * 4
        ),
    )
    return kern(onehot_flat, mh0_flat, mh1_flat, fm_emb, fm_w_flat)


def _tc_body(oh_ref, mh_ref, fw_ref, dn_ref, s416_ref, s32_ref,
             w0o_ref, w0m_ref, w0d_ref, b0_ref, w1_ref, b1_ref,
             fwh_ref, cw2_ref, cwn_ref, cb_ref, out_ref):
    f32 = jnp.float32

    def dot(a, b):
        return lax.dot_general(a, b, (((1,), (0,)), ((), ())),
                               preferred_element_type=f32)

    x = oh_ref[...]
    m = mh_ref[...]
    fw = fw_ref[...]
    dn = dn_ref[...]
    s = dot(x, s416_ref[...]) + dot(m, s32_ref[...])
    sq = dot(x * x, s416_ref[...]) + dot(m * m, s32_ref[...])
    fm2 = 0.5 * (s * s - sq)
    h0 = jnp.maximum(
        dot(x, w0o_ref[...]) + dot(m, w0m_ref[...]) + dot(dn, w0d_ref[...])
        + b0_ref[...], 0.0)
    h1 = jnp.maximum(dot(h0, w1_ref[...]) + b1_ref[...], 0.0)
    z = dot(fw, fwh_ref[...]) + dot(fm2, cw2_ref[...]) + dot(h1, cwn_ref[...])
    z = z + cb_ref[...]
    out_ref[...] = 1.0 / (1.0 + jnp.exp(-z))


def _tc_head(oh_flat, mh_flat, fw_vals, dense, s416, s32, w0o, w0m, w0d,
             b0, w1, b1, fwh, cw2, cwn, cb):
    bb = 2048
    grid = (_B // bb,)

    def row_block(minor):
        return pl.BlockSpec((bb, minor), lambda i: (i, 0))

    def full_block(shape):
        return pl.BlockSpec(shape, lambda i: (0, 0))

    return pl.pallas_call(
        _tc_body,
        grid=grid,
        in_specs=[
            row_block(_OH * _E),
            row_block(2 * _E),
            row_block(_OH),
            row_block(13),
            full_block((_OH * _E, _E)),
            full_block((2 * _E, _E)),
            full_block((_OH * _E, 64)),
            full_block((2 * _E, 64)),
            full_block((13, 64)),
            full_block((1, 64)),
            full_block((64, 12)),
            full_block((1, 12)),
            full_block((_OH, 1)),
            full_block((_E, 1)),
            full_block((12, 1)),
            full_block((1, 1)),
        ],
        out_specs=pl.BlockSpec((bb, 1), lambda i: (i, 0)),
        out_shape=jax.ShapeDtypeStruct((_B, 1), jnp.float32),
    )(oh_flat, mh_flat, fw_vals, dense, s416, s32, w0o, w0m, w0d, b0, w1,
      b1, fwh, cw2, cwn, cb)


def kernel(dense, onehot, multihot_0, multihot_1, fm_w, fm_emb,
           nn_w0, nn_b0, nn_w1, nn_b1, concat_w, concat_b):
    tab_lin = _linearize_table(fm_emb.T)
    oh_rows, mh_mean, fw_vals = _sc_gather(
        onehot.reshape(-1).astype(jnp.int32),
        multihot_0.reshape(-1).astype(jnp.int32),
        multihot_1.reshape(-1).astype(jnp.int32),
        tab_lin.reshape(_V, _E),
        fm_w.reshape(-1),
    )
    oh_flat = oh_rows.reshape(_B, _OH * _E)
    mh_flat = mh_mean.reshape(_B, 2 * _E)
    fw_flat = fw_vals.reshape(_B, _OH)

    eye = jnp.eye(_E, dtype=jnp.float32)
    s416 = jnp.tile(eye, (_OH, 1))
    s32 = jnp.tile(eye, (2, 1))
    w0o = nn_w0[: _OH * _E]
    w0m = nn_w0[_OH * _E: (_OH + 2) * _E]
    w0d = nn_w0[(_OH + 2) * _E:]
    fwh = jnp.full((_OH, 1), concat_w[0, 0], dtype=jnp.float32)
    cw2 = concat_w[1: 1 + _E]
    cwn = concat_w[1 + _E:]
    return _tc_head(oh_flat, mh_flat, fw_flat, dense, s416, s32, w0o, w0m,
                    w0d, nn_b0.reshape(1, 64), nn_w1, nn_b1.reshape(1, 12),
                    fwh, cw2, cwn, concat_b.reshape(1, 1))


# speedup vs baseline: 4.6454x; 1.0007x over previous
import functools

import jax
import jax.numpy as jnp
from jax import lax
from jax.experimental import pallas as pl
from jax.experimental.pallas import tpu as pltpu
from jax.experimental.pallas import tpu_sc as plsc

_B = 16384
_V = 1000000
_E = 16
_OH = 26
_MH = 20

_NC = 2
_NS = 16
_NW = _NC * _NS
_BPW = _B // _NW
_C = 32
_NCHUNK = _BPW // _C


_TBLK = 1024
_TFULL = _V // _TBLK
_TTAIL = 640


def _tr_body(tabT, tailT, out1d, in0, in1, tin, ob0, ob1, si0, si1, so0, so1):
    wid = lax.axis_index("s") * _NC + lax.axis_index("c")
    iota = lax.iota(jnp.int32, 16)
    f_idx = [iota * 16 + f for f in range(16)]
    inbufs = (in0, in1)
    obufs = (ob0, ob1)
    isems = (si0, si1)
    osems = (so0, so1)

    def issue_in(blk, buf, sem):
        @pl.when(blk < _TFULL)
        def _():
            pltpu.async_copy(tabT.at[:, pl.ds(blk * _TBLK, _TBLK)], buf, sem)

    def scatter_block(inbuf, ob):
        def g_body(g, carry):
            for f in range(16):
                v = inbuf[f, pl.ds(g * 16, 16)]
                plsc.store_scatter(ob, [f_idx[f] + g * 256], v)
            return carry
        lax.fori_loop(0, _TBLK // 16, g_body, 0)

    issue_in(wid, in0, si0)
    issue_in(wid + _NW, in1, si1)

    def outer(i2, carry):
        for b in range(2):
            i = 2 * i2 + b
            blk = wid + _NW * i

            @pl.when(blk < _TFULL)
            def _():
                pltpu.make_async_copy(
                    tabT.at[:, pl.ds(0, _TBLK)], inbufs[b], isems[b]).wait()

                @pl.when(i2 >= 1)
                def _():
                    pltpu.make_async_copy(
                        obufs[b], out1d.at[pl.ds(0, _TBLK * 16)],
                        osems[b]).wait()

                scatter_block(inbufs[b], obufs[b])
                pltpu.async_copy(
                    obufs[b], out1d.at[pl.ds(blk * _TBLK * 16, _TBLK * 16)],
                    osems[b])
                issue_in(blk + 2 * _NW, inbufs[b], isems[b])
        return carry

    lax.fori_loop(0, 16, outer, 0)
    for b in range(2):
        pltpu.make_async_copy(
            obufs[b], out1d.at[pl.ds(0, _TBLK * 16)], osems[b]).wait()

    @pl.when(wid == 0)
    def _():
        pltpu.sync_copy(tailT, tin)

        def tg_body(g, carry):
            for f in range(16):
                v = tin[f, pl.ds(g * 16, 16)]
                plsc.store_scatter(ob0, [f_idx[f] + g * 256], v)
            return carry
        lax.fori_loop(0, _TTAIL // 16, tg_body, 0)
        pltpu.sync_copy(
            ob0.at[pl.ds(0, _TTAIL * 16)],
            out1d.at[pl.ds((_V - _TTAIL) * 16, _TTAIL * 16)])


def _linearize_table(tabT):
    kern = pl.kernel(
        _tr_body,
        out_type=[jax.ShapeDtypeStruct((_V * _E,), jnp.float32)],
        mesh=plsc.VectorSubcoreMesh(core_axis_name="c", subcore_axis_name="s"),
        compiler_params=pltpu.CompilerParams(
            use_tc_tiling_on_sc=True, needs_layout_passes=False),
        scratch_types=[
            pltpu.VMEM((16, _TBLK), jnp.float32),
            pltpu.VMEM((16, _TBLK), jnp.float32),
            pltpu.VMEM((16, _TTAIL), jnp.float32),
            pltpu.VMEM((_TBLK * 16,), jnp.float32),
            pltpu.VMEM((_TBLK * 16,), jnp.float32),
            pltpu.SemaphoreType.DMA,
            pltpu.SemaphoreType.DMA,
            pltpu.SemaphoreType.DMA,
            pltpu.SemaphoreType.DMA,
        ],
    )
    return kern(tabT, tabT[:, _V - _TTAIL:])[0]


def _sc_body(oh_hbm, mh0_hbm, mh1_hbm, emb_hbm, fmw_hbm,
             out_oh, out_mh, out_fmw,
             oh_idx0, oh_idx1, mh0_idx0, mh0_idx1, mh1_idx0, mh1_idx1,
             oh_rows0, oh_rows1, mh0_rows0, mh0_rows1, mh1_rows0, mh1_rows1,
             fmw0, fmw1, means0, means1, sg0, sg1, so0, so1):
    wid = lax.axis_index("s") * _NC + lax.axis_index("c")
    oh_idx = (oh_idx0, oh_idx1)
    mh0_idx = (mh0_idx0, mh0_idx1)
    mh1_idx = (mh1_idx0, mh1_idx1)
    oh_rows = (oh_rows0, oh_rows1)
    mh0_rows = (mh0_rows0, mh0_rows1)
    mh1_rows = (mh1_rows0, mh1_rows1)
    fmw = (fmw0, fmw1)
    means = (means0, means1)
    sg = (sg0, sg1)
    so = (so0, so1)

    def stage_in(c, b):
        base = wid * _BPW + c * _C
        pltpu.sync_copy(oh_hbm.at[pl.ds(base * _OH, _C * _OH)], oh_idx[b])
        pltpu.sync_copy(mh0_hbm.at[pl.ds(base * _MH, _C * _MH)], mh0_idx[b])
        pltpu.sync_copy(mh1_hbm.at[pl.ds(base * _MH, _C * _MH)], mh1_idx[b])
        pltpu.async_copy(emb_hbm.at[oh_idx[b]], oh_rows[b], sg[b])
        pltpu.async_copy(emb_hbm.at[mh0_idx[b]], mh0_rows[b], sg[b])
        pltpu.async_copy(emb_hbm.at[mh1_idx[b]], mh1_rows[b], sg[b])
        pltpu.async_copy(fmw_hbm.at[oh_idx[b]], fmw[b], sg[b])

    def wait_gathers(b):
        pltpu.make_async_copy(emb_hbm.at[oh_idx[b]], oh_rows[b], sg[b]).wait()
        pltpu.make_async_copy(emb_hbm.at[mh0_idx[b]], mh0_rows[b],
                              sg[b]).wait()
        pltpu.make_async_copy(emb_hbm.at[mh1_idx[b]], mh1_rows[b],
                              sg[b]).wait()
        pltpu.make_async_copy(fmw_hbm.at[oh_idx[b]], fmw[b], sg[b]).wait()

    def wait_outs(b):
        pltpu.make_async_copy(oh_rows[b], out_oh.at[pl.ds(0, _C * _OH)],
                              so[b]).wait()
        pltpu.make_async_copy(means[b], out_mh.at[pl.ds(0, _C * 2 * _E)],
                              so[b]).wait()
        pltpu.make_async_copy(fmw[b], out_fmw.at[pl.ds(0, _C * _OH)],
                              so[b]).wait()

    stage_in(0, 0)
    stage_in(1, 1)

    def outer(c2, carry):
        for b in range(2):
            c = 2 * c2 + b
            base = wid * _BPW + c * _C
            wait_gathers(b)

            def mean_body(i, carry2):
                off = i * _MH
                acc0 = mh0_rows[b][off]
                acc1 = mh1_rows[b][off]
                for j in range(1, _MH):
                    acc0 = acc0 + mh0_rows[b][off + j]
                    acc1 = acc1 + mh1_rows[b][off + j]
                means[b][pl.ds(i * 2 * _E, _E)] = acc0 * (1.0 / _MH)
                means[b][pl.ds(i * 2 * _E + _E, _E)] = acc1 * (1.0 / _MH)
                return carry2

            lax.fori_loop(0, _C, mean_body, 0, unroll=2)

            pltpu.async_copy(oh_rows[b],
                             out_oh.at[pl.ds(base * _OH, _C * _OH)], so[b])
            pltpu.async_copy(means[b],
                             out_mh.at[pl.ds(base * 2 * _E, _C * 2 * _E)],
                             so[b])
            pltpu.async_copy(fmw[b],
                             out_fmw.at[pl.ds(base * _OH, _C * _OH)], so[b])
            wait_outs(b)

            @pl.when(c + 2 < _NCHUNK)
            def _():
                stage_in(c + 2, b)
        return carry

    lax.fori_loop(0, _NCHUNK // 2, outer, 0)


def _sc_gather(onehot_flat, mh0_flat, mh1_flat, fm_emb, fm_w_flat):
    kern = pl.kernel(
        _sc_body,
        out_type=[
            jax.ShapeDtypeStruct((_B * _OH, _E), jnp.float32),
            jax.ShapeDtypeStruct((_B * 2 * _E,), jnp.float32),
            jax.ShapeDtypeStruct((_B * _OH,), jnp.float32),
        ],
        mesh=plsc.VectorSubcoreMesh(core_axis_name="c", subcore_axis_name="s"),
        compiler_params=pltpu.CompilerParams(use_tc_tiling_on_sc=False),
        scratch_types=(
            [pltpu.VMEM((_C * _OH,), jnp.int32)] * 2
            + [pltpu.VMEM((_C * _MH,), jnp.int32)] * 4
            + [pltpu.VMEM((_C * _OH, _E), jnp.float32)] * 2
            + [pltpu.VMEM((_C * _MH, _E), jnp.float32)] * 4
            + [pltpu.VMEM((_C * _OH,), jnp.float32)] * 2
            + [pltpu.VMEM((_C * 2 * _E,), jnp.float32)] * 2
            + [pltpu.SemaphoreType.DMA] * 4
        ),
    )
    return kern(onehot_flat, mh0_flat, mh1_flat, fm_emb, fm_w_flat)


def _tc_body(oh_ref, mh_ref, fw_ref, dn_ref, s416_ref, s32_ref,
             w0o_ref, w0m_ref, w0d_ref, b0_ref, w1_ref, b1_ref,
             fwh_ref, cw2_ref, cwn_ref, cb_ref, out_ref):
    f32 = jnp.float32
    bf = jnp.bfloat16

    def dot(a, b):
        return lax.dot_general(a, b, (((1,), (0,)), ((), ())),
                               preferred_element_type=f32)

    def dotbf(a, b):
        return lax.dot_general(a.astype(bf), b.astype(bf),
                               (((1,), (0,)), ((), ())),
                               preferred_element_type=f32)

    x = oh_ref[...]
    m = mh_ref[...]
    fw = fw_ref[...]
    dn = dn_ref[...]
    s = dotbf(x, s416_ref[...]) + dotbf(m, s32_ref[...])
    sq = dot(x * x, s416_ref[...]) + dot(m * m, s32_ref[...])
    fm2 = 0.5 * (s * s - sq)
    h0 = jnp.maximum(
        dotbf(x, w0o_ref[...]) + dotbf(m, w0m_ref[...])
        + dotbf(dn, w0d_ref[...]) + b0_ref[...], 0.0)
    h1 = jnp.maximum(dotbf(h0, w1_ref[...]) + b1_ref[...], 0.0)
    z = dot(fw, fwh_ref[...]) + dot(fm2, cw2_ref[...]) + dot(h1, cwn_ref[...])
    z = z + cb_ref[...]
    out_ref[...] = 1.0 / (1.0 + jnp.exp(-z))


def _tc_head(oh_flat, mh_flat, fw_vals, dense, s416, s32, w0o, w0m, w0d,
             b0, w1, b1, fwh, cw2, cwn, cb):
    bb = 2048
    grid = (_B // bb,)

    def row_block(minor):
        return pl.BlockSpec((bb, minor), lambda i: (i, 0))

    def full_block(shape):
        return pl.BlockSpec(shape, lambda i: (0, 0))

    return pl.pallas_call(
        _tc_body,
        grid=grid,
        in_specs=[
            row_block(_OH * _E),
            row_block(2 * _E),
            row_block(_OH),
            row_block(13),
            full_block((_OH * _E, _E)),
            full_block((2 * _E, _E)),
            full_block((_OH * _E, 64)),
            full_block((2 * _E, 64)),
            full_block((13, 64)),
            full_block((1, 64)),
            full_block((64, 12)),
            full_block((1, 12)),
            full_block((_OH, 1)),
            full_block((_E, 1)),
            full_block((12, 1)),
            full_block((1, 1)),
        ],
        out_specs=pl.BlockSpec((bb, 1), lambda i: (i, 0)),
        out_shape=jax.ShapeDtypeStruct((_B, 1), jnp.float32),
    )(oh_flat, mh_flat, fw_vals, dense, s416, s32, w0o, w0m, w0d, b0, w1,
      b1, fwh, cw2, cwn, cb)


def kernel(dense, onehot, multihot_0, multihot_1, fm_w, fm_emb,
           nn_w0, nn_b0, nn_w1, nn_b1, concat_w, concat_b):
    tab_lin = _linearize_table(fm_emb.T)
    oh_rows, mh_mean, fw_vals = _sc_gather(
        onehot.reshape(-1).astype(jnp.int32),
        multihot_0.reshape(-1).astype(jnp.int32),
        multihot_1.reshape(-1).astype(jnp.int32),
        tab_lin.reshape(_V, _E),
        fm_w.reshape(-1),
    )
    oh_flat = oh_rows.reshape(_B, _OH * _E)
    mh_flat = mh_mean.reshape(_B, 2 * _E)
    fw_flat = fw_vals.reshape(_B, _OH)

    eye = jnp.eye(_E, dtype=jnp.float32)
    s416 = jnp.tile(eye, (_OH, 1))
    s32 = jnp.tile(eye, (2, 1))
    w0o = nn_w0[: _OH * _E]
    w0m = nn_w0[_OH * _E: (_OH + 2) * _E]
    w0d = nn_w0[(_OH + 2) * _E:]
    fwh = jnp.full((_OH, 1), concat_w[0, 0], dtype=jnp.float32)
    cw2 = concat_w[1: 1 + _E]
    cwn = concat_w[1 + _E:]
    return _tc_head(oh_flat, mh_flat, fw_flat, dense, s416, s32, w0o, w0m,
                    w0d, nn_b0.reshape(1, 64), nn_w1, nn_b1.reshape(1, 12),
                    fwh, cw2, cwn, concat_b.reshape(1, 1))


# speedup vs baseline: 4.9000x; 1.0548x over previous
import functools

import jax
import jax.numpy as jnp
from jax import lax
from jax.experimental import pallas as pl
from jax.experimental.pallas import tpu as pltpu
from jax.experimental.pallas import tpu_sc as plsc

_B = 16384
_V = 1000000
_E = 16
_OH = 26
_MH = 20

_NC = 2
_NS = 16
_NW = _NC * _NS
_BPW = _B // _NW
_C = 32
_NCHUNK = _BPW // _C


_TBLK = 1024
_TFULL = _V // _TBLK
_TTAIL = 640


def _tr_body(tabT, tailT, out1d, in0, in1, tin, ob0, ob1, si0, si1, so0, so1):
    wid = lax.axis_index("s") * _NC + lax.axis_index("c")
    iota = lax.iota(jnp.int32, 16)
    f_idx = [iota * 16 + f for f in range(16)]
    inbufs = (in0, in1)
    obufs = (ob0, ob1)
    isems = (si0, si1)
    osems = (so0, so1)

    def issue_in(blk, buf, sem):
        @pl.when(blk < _TFULL)
        def _():
            pltpu.async_copy(tabT.at[:, pl.ds(blk * _TBLK, _TBLK)], buf, sem)

    def scatter_block(inbuf, ob):
        def g_body(g, carry):
            for f in range(16):
                v = inbuf[f, pl.ds(g * 16, 16)]
                plsc.store_scatter(ob, [f_idx[f] + g * 256], v)
            return carry
        lax.fori_loop(0, _TBLK // 16, g_body, 0)

    issue_in(wid, in0, si0)
    issue_in(wid + _NW, in1, si1)

    def outer(i2, carry):
        for b in range(2):
            i = 2 * i2 + b
            blk = wid + _NW * i

            @pl.when(blk < _TFULL)
            def _():
                pltpu.make_async_copy(
                    tabT.at[:, pl.ds(0, _TBLK)], inbufs[b], isems[b]).wait()

                @pl.when(i2 >= 1)
                def _():
                    pltpu.make_async_copy(
                        obufs[b], out1d.at[pl.ds(0, _TBLK * 16)],
                        osems[b]).wait()

                scatter_block(inbufs[b], obufs[b])
                pltpu.async_copy(
                    obufs[b], out1d.at[pl.ds(blk * _TBLK * 16, _TBLK * 16)],
                    osems[b])
                issue_in(blk + 2 * _NW, inbufs[b], isems[b])
        return carry

    lax.fori_loop(0, 16, outer, 0)
    for b in range(2):
        pltpu.make_async_copy(
            obufs[b], out1d.at[pl.ds(0, _TBLK * 16)], osems[b]).wait()

    @pl.when(wid == 0)
    def _():
        pltpu.sync_copy(tailT, tin)

        def tg_body(g, carry):
            for f in range(16):
                v = tin[f, pl.ds(g * 16, 16)]
                plsc.store_scatter(ob0, [f_idx[f] + g * 256], v)
            return carry
        lax.fori_loop(0, _TTAIL // 16, tg_body, 0)
        pltpu.sync_copy(
            ob0.at[pl.ds(0, _TTAIL * 16)],
            out1d.at[pl.ds((_V - _TTAIL) * 16, _TTAIL * 16)])


def _linearize_table(tabT):
    kern = pl.kernel(
        _tr_body,
        out_type=[jax.ShapeDtypeStruct((_V * _E,), jnp.float32)],
        mesh=plsc.VectorSubcoreMesh(core_axis_name="c", subcore_axis_name="s"),
        compiler_params=pltpu.CompilerParams(
            use_tc_tiling_on_sc=True, needs_layout_passes=False),
        scratch_types=[
            pltpu.VMEM((16, _TBLK), jnp.float32),
            pltpu.VMEM((16, _TBLK), jnp.float32),
            pltpu.VMEM((16, _TTAIL), jnp.float32),
            pltpu.VMEM((_TBLK * 16,), jnp.float32),
            pltpu.VMEM((_TBLK * 16,), jnp.float32),
            pltpu.SemaphoreType.DMA,
            pltpu.SemaphoreType.DMA,
            pltpu.SemaphoreType.DMA,
            pltpu.SemaphoreType.DMA,
        ],
    )
    return kern(tabT, tabT[:, _V - _TTAIL:])[0]


_BH = _B // 2
_BPWH = _BH // _NW
_NCHUNKH = _BPWH // _C


def _sc_body(h, oh_hbm, mh0_hbm, mh1_hbm, emb_hbm, fmw_hbm,
             out_oh, out_mh, out_fmw,
             oh_idx0, oh_idx1, mh0_idx0, mh0_idx1, mh1_idx0, mh1_idx1,
             oh_rows0, oh_rows1, mh0_rows0, mh0_rows1, mh1_rows0, mh1_rows1,
             fmw0, fmw1, means0, means1, sg0, sg1, so0, so1):
    wid = lax.axis_index("s") * _NC + lax.axis_index("c")
    oh_idx = (oh_idx0, oh_idx1)
    mh0_idx = (mh0_idx0, mh0_idx1)
    mh1_idx = (mh1_idx0, mh1_idx1)
    oh_rows = (oh_rows0, oh_rows1)
    mh0_rows = (mh0_rows0, mh0_rows1)
    mh1_rows = (mh1_rows0, mh1_rows1)
    fmw = (fmw0, fmw1)
    means = (means0, means1)
    sg = (sg0, sg1)
    so = (so0, so1)

    def stage_in(c, b):
        base = h * _BH + wid * _BPWH + c * _C
        pltpu.sync_copy(oh_hbm.at[pl.ds(base * _OH, _C * _OH)], oh_idx[b])
        pltpu.sync_copy(mh0_hbm.at[pl.ds(base * _MH, _C * _MH)], mh0_idx[b])
        pltpu.sync_copy(mh1_hbm.at[pl.ds(base * _MH, _C * _MH)], mh1_idx[b])
        pltpu.async_copy(emb_hbm.at[oh_idx[b]], oh_rows[b], sg[b])
        pltpu.async_copy(emb_hbm.at[mh0_idx[b]], mh0_rows[b], sg[b])
        pltpu.async_copy(emb_hbm.at[mh1_idx[b]], mh1_rows[b], sg[b])
        pltpu.async_copy(fmw_hbm.at[oh_idx[b]], fmw[b], sg[b])

    def wait_gathers(b):
        pltpu.make_async_copy(emb_hbm.at[oh_idx[b]], oh_rows[b], sg[b]).wait()
        pltpu.make_async_copy(emb_hbm.at[mh0_idx[b]], mh0_rows[b],
                              sg[b]).wait()
        pltpu.make_async_copy(emb_hbm.at[mh1_idx[b]], mh1_rows[b],
                              sg[b]).wait()
        pltpu.make_async_copy(fmw_hbm.at[oh_idx[b]], fmw[b], sg[b]).wait()

    def wait_outs(b):
        pltpu.make_async_copy(oh_rows[b], out_oh.at[pl.ds(0, _C * _OH)],
                              so[b]).wait()
        pltpu.make_async_copy(means[b], out_mh.at[pl.ds(0, _C * 2 * _E)],
                              so[b]).wait()
        pltpu.make_async_copy(fmw[b], out_fmw.at[pl.ds(0, _C * _OH)],
                              so[b]).wait()

    stage_in(0, 0)
    stage_in(1, 1)

    def outer(c2, carry):
        for b in range(2):
            c = 2 * c2 + b
            base = wid * _BPWH + c * _C
            wait_gathers(b)

            def mean_body(i, carry2):
                off = i * _MH
                acc0 = mh0_rows[b][off]
                acc1 = mh1_rows[b][off]
                for j in range(1, _MH):
                    acc0 = acc0 + mh0_rows[b][off + j]
                    acc1 = acc1 + mh1_rows[b][off + j]
                means[b][pl.ds(i * 2 * _E, _E)] = acc0 * (1.0 / _MH)
                means[b][pl.ds(i * 2 * _E + _E, _E)] = acc1 * (1.0 / _MH)
                return carry2

            lax.fori_loop(0, _C, mean_body, 0, unroll=2)

            pltpu.async_copy(oh_rows[b],
                             out_oh.at[pl.ds(base * _OH, _C * _OH)], so[b])
            pltpu.async_copy(means[b],
                             out_mh.at[pl.ds(base * 2 * _E, _C * 2 * _E)],
                             so[b])
            pltpu.async_copy(fmw[b],
                             out_fmw.at[pl.ds(base * _OH, _C * _OH)], so[b])
            wait_outs(b)

            @pl.when(c + 2 < _NCHUNKH)
            def _():
                stage_in(c + 2, b)
        return carry

    lax.fori_loop(0, _NCHUNKH // 2, outer, 0)


def _sc_gather(h, onehot_flat, mh0_flat, mh1_flat, fm_emb, fm_w_flat):
    kern = pl.kernel(
        functools.partial(_sc_body, h),
        out_type=[
            jax.ShapeDtypeStruct((_BH * _OH, _E), jnp.float32),
            jax.ShapeDtypeStruct((_BH * 2 * _E,), jnp.float32),
            jax.ShapeDtypeStruct((_BH * _OH,), jnp.float32),
        ],
        mesh=plsc.VectorSubcoreMesh(core_axis_name="c", subcore_axis_name="s"),
        compiler_params=pltpu.CompilerParams(use_tc_tiling_on_sc=False),
        scratch_types=(
            [pltpu.VMEM((_C * _OH,), jnp.int32)] * 2
            + [pltpu.VMEM((_C * _MH,), jnp.int32)] * 4
            + [pltpu.VMEM((_C * _OH, _E), jnp.float32)] * 2
            + [pltpu.VMEM((_C * _MH, _E), jnp.float32)] * 4
            + [pltpu.VMEM((_C * _OH,), jnp.float32)] * 2
            + [pltpu.VMEM((_C * 2 * _E,), jnp.float32)] * 2
            + [pltpu.SemaphoreType.DMA] * 4
        ),
    )
    return kern(onehot_flat, mh0_flat, mh1_flat, fm_emb, fm_w_flat)


def _tc_body(oh_ref, mh_ref, fw_ref, dn_ref, s416_ref, s32_ref,
             w0o_ref, w0m_ref, w0d_ref, b0_ref, w1_ref, b1_ref,
             fwh_ref, cw2_ref, cwn_ref, cb_ref, out_ref):
    f32 = jnp.float32
    bf = jnp.bfloat16

    def dot(a, b):
        return lax.dot_general(a, b, (((1,), (0,)), ((), ())),
                               preferred_element_type=f32)

    def dotbf(a, b):
        return lax.dot_general(a.astype(bf), b.astype(bf),
                               (((1,), (0,)), ((), ())),
                               preferred_element_type=f32)

    x = oh_ref[...]
    m = mh_ref[...]
    fw = fw_ref[...]
    dn = dn_ref[...]
    s = dotbf(x, s416_ref[...]) + dotbf(m, s32_ref[...])
    sq = dot(x * x, s416_ref[...]) + dot(m * m, s32_ref[...])
    fm2 = 0.5 * (s * s - sq)
    h0 = jnp.maximum(
        dotbf(x, w0o_ref[...]) + dotbf(m, w0m_ref[...])
        + dotbf(dn, w0d_ref[...]) + b0_ref[...], 0.0)
    h1 = jnp.maximum(dotbf(h0, w1_ref[...]) + b1_ref[...], 0.0)
    z = dot(fw, fwh_ref[...]) + dot(fm2, cw2_ref[...]) + dot(h1, cwn_ref[...])
    z = z + cb_ref[...]
    out_ref[...] = 1.0 / (1.0 + jnp.exp(-z))


def _tc_head(oh_flat, mh_flat, fw_vals, dense, s416, s32, w0o, w0m, w0d,
             b0, w1, b1, fwh, cw2, cwn, cb):
    bb = 2048
    grid = (_BH // bb,)

    def row_block(minor):
        return pl.BlockSpec((bb, minor), lambda i: (i, 0))

    def full_block(shape):
        return pl.BlockSpec(shape, lambda i: (0, 0))

    return pl.pallas_call(
        _tc_body,
        grid=grid,
        in_specs=[
            row_block(_OH * _E),
            row_block(2 * _E),
            row_block(_OH),
            row_block(13),
            full_block((_OH * _E, _E)),
            full_block((2 * _E, _E)),
            full_block((_OH * _E, 64)),
            full_block((2 * _E, 64)),
            full_block((13, 64)),
            full_block((1, 64)),
            full_block((64, 12)),
            full_block((1, 12)),
            full_block((_OH, 1)),
            full_block((_E, 1)),
            full_block((12, 1)),
            full_block((1, 1)),
        ],
        out_specs=pl.BlockSpec((bb, 1), lambda i: (i, 0)),
        out_shape=jax.ShapeDtypeStruct((_BH, 1), jnp.float32),
    )(oh_flat, mh_flat, fw_vals, dense, s416, s32, w0o, w0m, w0d, b0, w1,
      b1, fwh, cw2, cwn, cb)


def kernel(dense, onehot, multihot_0, multihot_1, fm_w, fm_emb,
           nn_w0, nn_b0, nn_w1, nn_b1, concat_w, concat_b):
    tab_lin = _linearize_table(fm_emb.T)
    eye = jnp.eye(_E, dtype=jnp.float32)
    s416 = jnp.tile(eye, (_OH, 1))
    s32 = jnp.tile(eye, (2, 1))
    w0o = nn_w0[: _OH * _E]
    w0m = nn_w0[_OH * _E: (_OH + 2) * _E]
    w0d = nn_w0[(_OH + 2) * _E:]
    fwh = jnp.full((_OH, 1), concat_w[0, 0], dtype=jnp.float32)
    cw2 = concat_w[1: 1 + _E]
    cwn = concat_w[1 + _E:]

    oh_i = onehot.reshape(-1).astype(jnp.int32)
    mh0_i = multihot_0.reshape(-1).astype(jnp.int32)
    mh1_i = multihot_1.reshape(-1).astype(jnp.int32)
    tab2d = tab_lin.reshape(_V, _E)
    fmw1d = fm_w.reshape(-1)

    outs = []
    for h in range(2):
        oh_rows, mh_mean, fw_vals = _sc_gather(
            h, oh_i, mh0_i, mh1_i, tab2d, fmw1d)
        outs.append(_tc_head(
            oh_rows.reshape(_BH, _OH * _E), mh_mean.reshape(_BH, 2 * _E),
            fw_vals.reshape(_BH, _OH),
            lax.slice_in_dim(dense, h * _BH, (h + 1) * _BH),
            s416, s32, w0o, w0m, w0d, nn_b0.reshape(1, 64), nn_w1,
            nn_b1.reshape(1, 12), fwh, cw2, cwn, concat_b.reshape(1, 1)))
    return jnp.concatenate(outs, axis=0)
